# Initial kernel scaffold; baseline (speedup 1.0000x reference)
import jax
import jax.numpy as jnp
from jax.experimental import pallas as pl


def kernel(x, edge_index, edge_attr, neW1, neb1, neW2, neb2, eeW1, eeb1, eeW2, eeb2, gW1, gb1, gW2, gb2, lng, lnb, ppW1, ppb1, ppW2, ppb2, ppW3, ppb3):
    raise NotImplementedError("write your pallas kernel here")


# speedup vs baseline: 2.9867x; 2.9867x over previous
import functools

import jax
import jax.numpy as jnp
from jax import lax
from jax.experimental import pallas as pl
from jax.experimental.pallas import tpu as pltpu
from jax.experimental.pallas import tpu_sc as plsc

N = 10000
E = 320000
NF, EF, NE, EE, GE, OF = 128, 16, 16, 16, 32, 128

NC, NS, L = 2, 16, 16
NW = NC * NS
EPW = E // NW
GCH = 2000
SCH = 2000
NPS = N // NS


def _leaky(v):
    return jnp.maximum(v, 0.01 * v)


def _node_body(x_ref, w1, b1, w2, b2, wx, o_ref):
    h = _leaky(jnp.dot(x_ref[...], w1[...], preferred_element_type=jnp.float32) + b1[...])
    h = _leaky(jnp.dot(h, w2[...], preferred_element_type=jnp.float32) + b2[...])
    o_ref[...] = jnp.dot(h, wx[...], preferred_element_type=jnp.float32)


def _msg_body(ea_ref, xp_ref, w1, b1, w2, b2, we, gb1, gw2, gb2, o_ref):
    h = _leaky(jnp.dot(ea_ref[...], w1[...], preferred_element_type=jnp.float32) + b1[...])
    h = _leaky(jnp.dot(h, w2[...], preferred_element_type=jnp.float32) + b2[...])
    h1 = _leaky(jnp.dot(h, we[...], preferred_element_type=jnp.float32)
                + xp_ref[...] + gb1[...])
    o_ref[...] = _leaky(jnp.dot(h1, gw2[...], preferred_element_type=jnp.float32) + gb2[...])


def _post_body(agg2_ref, x_ref, lng_a, lnb_a, lng_x, lnb_x,
               w1a, w1x, b1, w2, b2, w3, b3, o_ref):
    agg = agg2_ref[0] + agg2_ref[1]
    xb = x_ref[...]
    denom = float(GE + NF)
    mu = (jnp.sum(agg, axis=1, keepdims=True)
          + jnp.sum(xb, axis=1, keepdims=True)) / denom
    da = agg - mu
    dx = xb - mu
    var = (jnp.sum(da * da, axis=1, keepdims=True)
           + jnp.sum(dx * dx, axis=1, keepdims=True)) / denom
    inv = lax.rsqrt(var + 1e-5)
    ha = da * inv * lng_a[...] + lnb_a[...]
    hx = dx * inv * lng_x[...] + lnb_x[...]
    h = _leaky(jnp.dot(ha, w1a[...], preferred_element_type=jnp.float32)
               + jnp.dot(hx, w1x[...], preferred_element_type=jnp.float32) + b1[...])
    h = _leaky(jnp.dot(h, w2[...], preferred_element_type=jnp.float32) + b2[...])
    o_ref[...] = jnp.dot(h, w3[...], preferred_element_type=jnp.float32) + b3[...]


def _full(shape):
    return pl.BlockSpec(shape, lambda i: (0,) * len(shape))


def _gather_body(tab_hbm, idx_hbm, out_hbm, idx_v, rows_v, sem):
    wid = lax.axis_index("s") * NC + lax.axis_index("c")
    base = wid * EPW

    def body(i, carry):
        off = pl.multiple_of(base + i * GCH, 8)
        pltpu.sync_copy(idx_hbm.at[pl.ds(off, GCH)], idx_v)
        pltpu.async_copy(tab_hbm.at[idx_v], rows_v, sem).wait()
        pltpu.sync_copy(rows_v, out_hbm.at[pl.ds(off, GCH)])
        return carry

    lax.fori_loop(0, EPW // GCH, body, 0)


def _scatter_body(msg_hbm, dst_hbm, zeros_hbm, out_hbm,
                  idx_v, msg_v, tmp_v, acc_sh, sem):
    cid = lax.axis_index("c")
    sid = lax.axis_index("s")
    wid = sid * NC + cid
    pltpu.sync_copy(zeros_hbm.at[pl.ds(0, NPS)], tmp_v)
    pltpu.sync_copy(tmp_v, acc_sh.at[pl.ds(sid * NPS, NPS)])
    plsc.subcore_barrier()
    base = wid * EPW

    def body(i, carry):
        off = pl.multiple_of(base + i * SCH, 8)
        pltpu.sync_copy(dst_hbm.at[pl.ds(off, SCH)], idx_v)
        pltpu.sync_copy(msg_hbm.at[pl.ds(off, SCH)], msg_v)
        pltpu.sync_copy(msg_v, acc_sh.at[idx_v], add=True)
        return carry

    lax.fori_loop(0, EPW // SCH, body, 0)
    plsc.subcore_barrier()
    pltpu.sync_copy(acc_sh.at[pl.ds(sid * NPS, NPS)], tmp_v)
    pltpu.sync_copy(tmp_v, out_hbm.at[cid, pl.ds(sid * NPS, NPS)])


def _sc_gather(node_pre, src):
    call = pl.kernel(
        _gather_body,
        out_type=jax.ShapeDtypeStruct((E, NE), jnp.float32),
        mesh=plsc.VectorSubcoreMesh(core_axis_name="c", subcore_axis_name="s",
                                    num_cores=NC, num_subcores=NS),
        scratch_types=[
            pltpu.VMEM((GCH,), jnp.int32),
            pltpu.VMEM((GCH, NE), jnp.float32),
            pltpu.SemaphoreType.DMA,
        ],
        compiler_params=pltpu.CompilerParams(use_tc_tiling_on_sc=False),
    )
    return call(node_pre, src)


def _sc_scatter(msg, dst, zeros):
    call = pl.kernel(
        _scatter_body,
        out_type=jax.ShapeDtypeStruct((NC, N, GE), jnp.float32),
        mesh=plsc.VectorSubcoreMesh(core_axis_name="c", subcore_axis_name="s",
                                    num_cores=NC, num_subcores=NS),
        scratch_types=[
            pltpu.VMEM((SCH,), jnp.int32),
            pltpu.VMEM((SCH, GE), jnp.float32),
            pltpu.VMEM((NPS, GE), jnp.float32),
            pltpu.VMEM_SHARED((N, GE), jnp.float32),
            pltpu.SemaphoreType.DMA,
        ],
        compiler_params=pltpu.CompilerParams(use_tc_tiling_on_sc=False),
    )
    return call(msg, dst, zeros)


def kernel(x, edge_index, edge_attr, neW1, neb1, neW2, neb2, eeW1, eeb1,
           eeW2, eeb2, gW1, gb1, gW2, gb2, lng, lnb, ppW1, ppb1, ppW2, ppb2,
           ppW3, ppb3):
    src = edge_index[0].astype(jnp.int32)
    dst = edge_index[1].astype(jnp.int32)
    gW1e = gW1[:EE]
    gW1x = gW1[EE:]
    r = lambda v: v.reshape(1, -1)

    RN = 2000
    node_pre = pl.pallas_call(
        _node_body,
        grid=(N // RN,),
        in_specs=[
            pl.BlockSpec((RN, NF), lambda i: (i, 0)),
            _full((NF, NE)), _full((1, NE)), _full((NE, NE)), _full((1, NE)),
            _full((NE, NE)),
        ],
        out_specs=pl.BlockSpec((RN, NE), lambda i: (i, 0)),
        out_shape=jax.ShapeDtypeStruct((N, NE), jnp.float32),
    )(x, neW1, r(neb1), neW2, r(neb2), gW1x)

    x_pre = _sc_gather(node_pre, src)

    RE = 8000
    msg = pl.pallas_call(
        _msg_body,
        grid=(E // RE,),
        in_specs=[
            pl.BlockSpec((RE, EF), lambda i: (i, 0)),
            pl.BlockSpec((RE, NE), lambda i: (i, 0)),
            _full((EF, EE)), _full((1, EE)), _full((EE, EE)), _full((1, EE)),
            _full((EE, NE)), _full((1, NE)), _full((NE, GE)), _full((1, GE)),
        ],
        out_specs=pl.BlockSpec((RE, GE), lambda i: (i, 0)),
        out_shape=jax.ShapeDtypeStruct((E, GE), jnp.float32),
    )(edge_attr, x_pre, eeW1, r(eeb1), eeW2, r(eeb2), gW1e, r(gb1), gW2, r(gb2))

    zeros = jnp.zeros((NPS, GE), dtype=jnp.float32)
    agg2 = _sc_scatter(msg, dst, zeros)

    RP = 2000
    out = pl.pallas_call(
        _post_body,
        grid=(N // RP,),
        in_specs=[
            pl.BlockSpec((NC, RP, GE), lambda i: (0, i, 0)),
            pl.BlockSpec((RP, NF), lambda i: (i, 0)),
            _full((1, GE)), _full((1, GE)), _full((1, NF)), _full((1, NF)),
            _full((GE, GE)), _full((NF, GE)), _full((1, GE)),
            _full((GE, GE)), _full((1, GE)),
            _full((GE, OF)), _full((1, OF)),
        ],
        out_specs=pl.BlockSpec((RP, OF), lambda i: (i, 0)),
        out_shape=jax.ShapeDtypeStruct((N, OF), jnp.float32),
    )(agg2, x, r(lng[:GE]), r(lnb[:GE]), r(lng[GE:]), r(lnb[GE:]),
      ppW1[:GE], ppW1[GE:], r(ppb1), ppW2, r(ppb2), ppW3, r(ppb3))

    return out


# speedup vs baseline: 5.6074x; 1.8774x over previous
import functools

import jax
import jax.numpy as jnp
from jax import lax
from jax.experimental import pallas as pl
from jax.experimental.pallas import tpu as pltpu
from jax.experimental.pallas import tpu_sc as plsc

N = 10000
E = 320000
NF, EF, NE, EE, GE, OF = 128, 16, 16, 16, 32, 128

NC, NS, L = 2, 16, 16
NW = NC * NS
EPW = E // NW
GCH = 2000
SCH = 2000
NPS = N // NS


def _leaky(v):
    return jnp.maximum(v, 0.01 * v)


def _node_body(x_ref, w1, b1, w2, b2, wx, o_ref):
    h = _leaky(jnp.dot(x_ref[...], w1[...], preferred_element_type=jnp.float32) + b1[...])
    h = _leaky(jnp.dot(h, w2[...], preferred_element_type=jnp.float32) + b2[...])
    o_ref[...] = jnp.dot(h, wx[...], preferred_element_type=jnp.float32)


def _msg_body(ea_ref, xp_ref, w1, b1, w2, b2, we, gb1, gw2, gb2, o_ref):
    h = _leaky(jnp.dot(ea_ref[...], w1[...], preferred_element_type=jnp.float32) + b1[...])
    h = _leaky(jnp.dot(h, w2[...], preferred_element_type=jnp.float32) + b2[...])
    h1 = _leaky(jnp.dot(h, we[...], preferred_element_type=jnp.float32)
                + xp_ref[...] + gb1[...])
    o_ref[...] = _leaky(jnp.dot(h1, gw2[...], preferred_element_type=jnp.float32) + gb2[...])


def _post_body(agg2_ref, x_ref, lng_a, lnb_a, lng_x, lnb_x,
               w1a, w1x, b1, w2, b2, w3, b3, o_ref):
    agg = agg2_ref[0] + agg2_ref[1]
    xb = x_ref[...]
    denom = float(GE + NF)
    mu = (jnp.sum(agg, axis=1, keepdims=True)
          + jnp.sum(xb, axis=1, keepdims=True)) / denom
    da = agg - mu
    dx = xb - mu
    var = (jnp.sum(da * da, axis=1, keepdims=True)
           + jnp.sum(dx * dx, axis=1, keepdims=True)) / denom
    inv = lax.rsqrt(var + 1e-5)
    ha = da * inv * lng_a[...] + lnb_a[...]
    hx = dx * inv * lng_x[...] + lnb_x[...]
    h = _leaky(jnp.dot(ha, w1a[...], preferred_element_type=jnp.float32)
               + jnp.dot(hx, w1x[...], preferred_element_type=jnp.float32) + b1[...])
    h = _leaky(jnp.dot(h, w2[...], preferred_element_type=jnp.float32) + b2[...])
    o_ref[...] = jnp.dot(h, w3[...], preferred_element_type=jnp.float32) + b3[...]


def _full(shape):
    return pl.BlockSpec(shape, lambda i: (0,) * len(shape))


def _gather_body(tab_hbm, idx_hbm, out_hbm, idx_v, rows_v, sem):
    wid = lax.axis_index("s") * NC + lax.axis_index("c")
    base = wid * EPW

    def body(i, carry):
        off = pl.multiple_of(base + i * GCH, 8)
        pltpu.sync_copy(idx_hbm.at[pl.ds(off, GCH)], idx_v)
        pltpu.async_copy(tab_hbm.at[idx_v], rows_v, sem).wait()
        pltpu.sync_copy(rows_v, out_hbm.at[pl.ds(off, GCH)])
        return carry

    lax.fori_loop(0, EPW // GCH, body, 0)


def _scatter_body(msg_hbm, dst_hbm, zeros_hbm, out_hbm,
                  idx_v, msg_v, tmp_v, acc_sh, sem):
    cid = lax.axis_index("c")
    sid = lax.axis_index("s")
    wid = sid * NC + cid
    pltpu.sync_copy(zeros_hbm.at[pl.ds(0, NPS)], tmp_v)
    pltpu.sync_copy(tmp_v, acc_sh.at[pl.ds(sid * NPS, NPS)])
    plsc.subcore_barrier()
    base = wid * EPW

    def body(i, carry):
        off = pl.multiple_of(base + i * SCH, 8)
        pltpu.sync_copy(dst_hbm.at[pl.ds(off, SCH)], idx_v)
        pltpu.sync_copy(msg_hbm.at[pl.ds(off, SCH)], msg_v)
        pltpu.sync_copy(msg_v, acc_sh.at[idx_v], add=True)
        return carry

    lax.fori_loop(0, EPW // SCH, body, 0)
    plsc.subcore_barrier()
    pltpu.sync_copy(acc_sh.at[pl.ds(sid * NPS, NPS)], tmp_v)
    pltpu.sync_copy(tmp_v, out_hbm.at[cid, pl.ds(sid * NPS, NPS)])


def _sc_gather(node_pre, src):
    call = pl.kernel(
        _gather_body,
        out_type=jax.ShapeDtypeStruct((E, NE), jnp.float32),
        mesh=plsc.VectorSubcoreMesh(core_axis_name="c", subcore_axis_name="s",
                                    num_cores=NC, num_subcores=NS),
        scratch_types=[
            pltpu.VMEM((GCH,), jnp.int32),
            pltpu.VMEM((GCH, NE), jnp.float32),
            pltpu.SemaphoreType.DMA,
        ],
        compiler_params=pltpu.CompilerParams(use_tc_tiling_on_sc=False),
    )
    return call(node_pre, src)


def _sc_scatter(msg, dst, zeros):
    call = pl.kernel(
        _scatter_body,
        out_type=jax.ShapeDtypeStruct((NC, N, GE), jnp.float32),
        mesh=plsc.VectorSubcoreMesh(core_axis_name="c", subcore_axis_name="s",
                                    num_cores=NC, num_subcores=NS),
        scratch_types=[
            pltpu.VMEM((SCH,), jnp.int32),
            pltpu.VMEM((SCH, GE), jnp.float32),
            pltpu.VMEM((NPS, GE), jnp.float32),
            pltpu.VMEM_SHARED((N, GE), jnp.float32),
            pltpu.SemaphoreType.DMA,
        ],
        compiler_params=pltpu.CompilerParams(use_tc_tiling_on_sc=False),
    )
    return call(msg, dst, zeros)


def kernel(x, edge_index, edge_attr, neW1, neb1, neW2, neb2, eeW1, eeb1,
           eeW2, eeb2, gW1, gb1, gW2, gb2, lng, lnb, ppW1, ppb1, ppW2, ppb2,
           ppW3, ppb3):
    src = edge_index[0].astype(jnp.int32)
    dst = edge_index[1].astype(jnp.int32)
    gW1e = gW1[:EE]
    gW1x = gW1[EE:]
    r = lambda v: v.reshape(1, -1)

    RN = 2000
    node_pre = pl.pallas_call(
        _node_body,
        grid=(N // RN,),
        in_specs=[
            pl.BlockSpec((RN, NF), lambda i: (i, 0)),
            _full((NF, NE)), _full((1, NE)), _full((NE, NE)), _full((1, NE)),
            _full((NE, NE)),
        ],
        out_specs=pl.BlockSpec((RN, NE), lambda i: (i, 0)),
        out_shape=jax.ShapeDtypeStruct((N, NE), jnp.float32),
    )(x, neW1, r(neb1), neW2, r(neb2), gW1x)

    x_pre = _sc_gather(node_pre, src)

    P = 8
    EP = E // P
    eye = jnp.eye(P, dtype=jnp.float32)
    kb = lambda w: jnp.kron(eye, w)
    tb = lambda b: jnp.tile(b, P).reshape(1, -1)
    ea_p = edge_attr.reshape(EP, P * EF)
    xp_p = x_pre.reshape(EP, P * NE)

    REP = 2000
    msg_p = pl.pallas_call(
        _msg_body,
        grid=(EP // REP,),
        in_specs=[
            pl.BlockSpec((REP, P * EF), lambda i: (i, 0)),
            pl.BlockSpec((REP, P * NE), lambda i: (i, 0)),
            _full((P * EF, P * EE)), _full((1, P * EE)),
            _full((P * EE, P * EE)), _full((1, P * EE)),
            _full((P * EE, P * NE)), _full((1, P * NE)),
            _full((P * NE, P * GE)), _full((1, P * GE)),
        ],
        out_specs=pl.BlockSpec((REP, P * GE), lambda i: (i, 0)),
        out_shape=jax.ShapeDtypeStruct((EP, P * GE), jnp.float32),
    )(ea_p, xp_p, kb(eeW1), tb(eeb1), kb(eeW2), tb(eeb2),
      kb(gW1e), tb(gb1), kb(gW2), tb(gb2))
    msg = msg_p.reshape(E, GE)

    zeros = jnp.zeros((NPS, GE), dtype=jnp.float32)
    agg2 = _sc_scatter(msg, dst, zeros)

    RP = 2000
    out = pl.pallas_call(
        _post_body,
        grid=(N // RP,),
        in_specs=[
            pl.BlockSpec((NC, RP, GE), lambda i: (0, i, 0)),
            pl.BlockSpec((RP, NF), lambda i: (i, 0)),
            _full((1, GE)), _full((1, GE)), _full((1, NF)), _full((1, NF)),
            _full((GE, GE)), _full((NF, GE)), _full((1, GE)),
            _full((GE, GE)), _full((1, GE)),
            _full((GE, OF)), _full((1, OF)),
        ],
        out_specs=pl.BlockSpec((RP, OF), lambda i: (i, 0)),
        out_shape=jax.ShapeDtypeStruct((N, OF), jnp.float32),
    )(agg2, x, r(lng[:GE]), r(lnb[:GE]), r(lng[GE:]), r(lnb[GE:]),
      ppW1[:GE], ppW1[GE:], r(ppb1), ppW2, r(ppb2), ppW3, r(ppb3))

    return out


# speedup vs baseline: 6.1888x; 1.1037x over previous
import jax
import jax.numpy as jnp
from jax import lax
from jax.experimental import pallas as pl
from jax.experimental.pallas import tpu as pltpu
from jax.experimental.pallas import tpu_sc as plsc

N = 10000
E = 320000
NF, EF, NE, EE, GE, OF = 128, 16, 16, 16, 32, 128
HG = GE // 2

NC, NS, L = 2, 16, 16
NW = NC * NS
EPW = E // NW
GCH = 2000
SCH = 2000
NPS = N // NS

P = 8
EP = E // P


def _leaky(v):
    return jnp.maximum(v, 0.01 * v)


def _node_body(x_ref, w1, b1, w2, b2, wx, o_ref):
    h = _leaky(jnp.dot(x_ref[...], w1[...], preferred_element_type=jnp.float32) + b1[...])
    h = _leaky(jnp.dot(h, w2[...], preferred_element_type=jnp.float32) + b2[...])
    o_ref[...] = jnp.dot(h, wx[...], preferred_element_type=jnp.float32)


def _msg_body(ea_ref, xp_ref, w1, b1, w2, b2, we, gb1, gw2l, gb2l, gw2h, gb2h,
              olo_ref, ohi_ref):
    h = _leaky(jnp.dot(ea_ref[...], w1[...], preferred_element_type=jnp.float32) + b1[...])
    h = _leaky(jnp.dot(h, w2[...], preferred_element_type=jnp.float32) + b2[...])
    h1 = _leaky(jnp.dot(h, we[...], preferred_element_type=jnp.float32)
                + xp_ref[...] + gb1[...])
    olo_ref[...] = _leaky(jnp.dot(h1, gw2l[...], preferred_element_type=jnp.float32) + gb2l[...])
    ohi_ref[...] = _leaky(jnp.dot(h1, gw2h[...], preferred_element_type=jnp.float32) + gb2h[...])


def _post_body(alo_ref, ahi_ref, x_ref, lng_l, lnb_l, lng_h, lnb_h,
               lng_x, lnb_x, w1l, w1h, w1x, b1, w2, b2, w3, b3, o_ref):
    al = alo_ref[0] + alo_ref[1]
    ah = ahi_ref[0] + ahi_ref[1]
    xb = x_ref[...]
    denom = float(GE + NF)
    mu = (jnp.sum(al, axis=1, keepdims=True) + jnp.sum(ah, axis=1, keepdims=True)
          + jnp.sum(xb, axis=1, keepdims=True)) / denom
    dl = al - mu
    dh = ah - mu
    dx = xb - mu
    var = (jnp.sum(dl * dl, axis=1, keepdims=True)
           + jnp.sum(dh * dh, axis=1, keepdims=True)
           + jnp.sum(dx * dx, axis=1, keepdims=True)) / denom
    inv = lax.rsqrt(var + 1e-5)
    hl = dl * inv * lng_l[...] + lnb_l[...]
    hh = dh * inv * lng_h[...] + lnb_h[...]
    hx = dx * inv * lng_x[...] + lnb_x[...]
    h = _leaky(jnp.dot(hl, w1l[...], preferred_element_type=jnp.float32)
               + jnp.dot(hh, w1h[...], preferred_element_type=jnp.float32)
               + jnp.dot(hx, w1x[...], preferred_element_type=jnp.float32) + b1[...])
    h = _leaky(jnp.dot(h, w2[...], preferred_element_type=jnp.float32) + b2[...])
    o_ref[...] = jnp.dot(h, w3[...], preferred_element_type=jnp.float32) + b3[...]


def _full(shape):
    return pl.BlockSpec(shape, lambda i: (0,) * len(shape))


def _gather_body(tab_hbm, eidx_hbm, out_hbm, idx_v, rows_v, sem):
    wid = lax.axis_index("s") * NC + lax.axis_index("c")
    base = wid * EPW

    def body(i, carry):
        off = pl.multiple_of(base + i * GCH, 8)
        pltpu.sync_copy(eidx_hbm.at[0, pl.ds(off, GCH)], idx_v)
        pltpu.async_copy(tab_hbm.at[idx_v], rows_v, sem).wait()
        pltpu.sync_copy(rows_v, out_hbm.at[pl.ds(off, GCH)])
        return carry

    lax.fori_loop(0, EPW // GCH, body, 0)


def _scatter_body(mlo_hbm, mhi_hbm, eidx_hbm, zeros_hbm, olo_hbm, ohi_hbm,
                  idx_v, mlo_v, mhi_v, tmp_v, acclo_sh, acchi_sh, sem):
    cid = lax.axis_index("c")
    sid = lax.axis_index("s")
    wid = sid * NC + cid
    pltpu.sync_copy(zeros_hbm.at[pl.ds(0, NPS)], tmp_v)
    pltpu.sync_copy(tmp_v, acclo_sh.at[pl.ds(sid * NPS, NPS)])
    pltpu.sync_copy(tmp_v, acchi_sh.at[pl.ds(sid * NPS, NPS)])
    plsc.subcore_barrier()
    base = wid * EPW

    def body(i, carry):
        off = pl.multiple_of(base + i * SCH, 8)
        pltpu.sync_copy(eidx_hbm.at[1, pl.ds(off, SCH)], idx_v)
        pltpu.sync_copy(mlo_hbm.at[pl.ds(off, SCH)], mlo_v)
        pltpu.sync_copy(mhi_hbm.at[pl.ds(off, SCH)], mhi_v)
        pltpu.sync_copy(mlo_v, acclo_sh.at[idx_v], add=True)
        pltpu.sync_copy(mhi_v, acchi_sh.at[idx_v], add=True)
        return carry

    lax.fori_loop(0, EPW // SCH, body, 0)
    plsc.subcore_barrier()
    pltpu.sync_copy(acclo_sh.at[pl.ds(sid * NPS, NPS)], tmp_v)
    pltpu.sync_copy(tmp_v, olo_hbm.at[cid, pl.ds(sid * NPS, NPS)])
    pltpu.sync_copy(acchi_sh.at[pl.ds(sid * NPS, NPS)], tmp_v)
    pltpu.sync_copy(tmp_v, ohi_hbm.at[cid, pl.ds(sid * NPS, NPS)])


def _sc_gather(node_pre, edge_index):
    call = pl.kernel(
        _gather_body,
        out_type=jax.ShapeDtypeStruct((E, NE), jnp.float32),
        mesh=plsc.VectorSubcoreMesh(core_axis_name="c", subcore_axis_name="s",
                                    num_cores=NC, num_subcores=NS),
        scratch_types=[
            pltpu.VMEM((GCH,), jnp.int32),
            pltpu.VMEM((GCH, NE), jnp.float32),
            pltpu.SemaphoreType.DMA,
        ],
        compiler_params=pltpu.CompilerParams(use_tc_tiling_on_sc=False),
    )
    return call(node_pre, edge_index)


def _sc_scatter(msg_lo, msg_hi, edge_index, zeros):
    call = pl.kernel(
        _scatter_body,
        out_type=(jax.ShapeDtypeStruct((NC, N, HG), jnp.float32),
                  jax.ShapeDtypeStruct((NC, N, HG), jnp.float32)),
        mesh=plsc.VectorSubcoreMesh(core_axis_name="c", subcore_axis_name="s",
                                    num_cores=NC, num_subcores=NS),
        scratch_types=[
            pltpu.VMEM((SCH,), jnp.int32),
            pltpu.VMEM((SCH, HG), jnp.float32),
            pltpu.VMEM((SCH, HG), jnp.float32),
            pltpu.VMEM((NPS, HG), jnp.float32),
            pltpu.VMEM_SHARED((N, HG), jnp.float32),
            pltpu.VMEM_SHARED((N, HG), jnp.float32),
            pltpu.SemaphoreType.DMA,
        ],
        compiler_params=pltpu.CompilerParams(use_tc_tiling_on_sc=False),
    )
    return call(msg_lo, msg_hi, edge_index, zeros)


def kernel(x, edge_index, edge_attr, neW1, neb1, neW2, neb2, eeW1, eeb1,
           eeW2, eeb2, gW1, gb1, gW2, gb2, lng, lnb, ppW1, ppb1, ppW2, ppb2,
           ppW3, ppb3):
    eidx = edge_index.astype(jnp.int32)
    gW1e = gW1[:EE]
    gW1x = gW1[EE:]
    r = lambda v: v.reshape(1, -1)

    RN = 2000
    node_pre = pl.pallas_call(
        _node_body,
        grid=(N // RN,),
        in_specs=[
            pl.BlockSpec((RN, NF), lambda i: (i, 0)),
            _full((NF, NE)), _full((1, NE)), _full((NE, NE)), _full((1, NE)),
            _full((NE, NE)),
        ],
        out_specs=pl.BlockSpec((RN, NE), lambda i: (i, 0)),
        out_shape=jax.ShapeDtypeStruct((N, NE), jnp.float32),
    )(x, neW1, r(neb1), neW2, r(neb2), gW1x)

    x_pre = _sc_gather(node_pre, eidx)

    eye = jnp.eye(P, dtype=jnp.float32)
    kb = lambda w: jnp.kron(eye, w)
    tb = lambda b: jnp.tile(b, P).reshape(1, -1)
    ea_p = edge_attr.reshape(EP, P * EF)
    xp_p = x_pre.reshape(EP, P * NE)

    REP = 2000
    mlo_p, mhi_p = pl.pallas_call(
        _msg_body,
        grid=(EP // REP,),
        in_specs=[
            pl.BlockSpec((REP, P * EF), lambda i: (i, 0)),
            pl.BlockSpec((REP, P * NE), lambda i: (i, 0)),
            _full((P * EF, P * EE)), _full((1, P * EE)),
            _full((P * EE, P * EE)), _full((1, P * EE)),
            _full((P * EE, P * NE)), _full((1, P * NE)),
            _full((P * NE, P * HG)), _full((1, P * HG)),
            _full((P * NE, P * HG)), _full((1, P * HG)),
        ],
        out_specs=(pl.BlockSpec((REP, P * HG), lambda i: (i, 0)),
                   pl.BlockSpec((REP, P * HG), lambda i: (i, 0))),
        out_shape=(jax.ShapeDtypeStruct((EP, P * HG), jnp.float32),
                   jax.ShapeDtypeStruct((EP, P * HG), jnp.float32)),
    )(ea_p, xp_p, kb(eeW1), tb(eeb1), kb(eeW2), tb(eeb2),
      kb(gW1e), tb(gb1), kb(gW2[:, :HG]), tb(gb2[:HG]),
      kb(gW2[:, HG:]), tb(gb2[HG:]))

    zeros = jnp.zeros((NPS, HG), dtype=jnp.float32)
    alo, ahi = _sc_scatter(mlo_p.reshape(E, HG), mhi_p.reshape(E, HG),
                           eidx, zeros)

    RP = 2000
    out = pl.pallas_call(
        _post_body,
        grid=(N // RP,),
        in_specs=[
            pl.BlockSpec((NC, RP, HG), lambda i: (0, i, 0)),
            pl.BlockSpec((NC, RP, HG), lambda i: (0, i, 0)),
            pl.BlockSpec((RP, NF), lambda i: (i, 0)),
            _full((1, HG)), _full((1, HG)), _full((1, HG)), _full((1, HG)),
            _full((1, NF)), _full((1, NF)),
            _full((HG, GE)), _full((HG, GE)), _full((NF, GE)), _full((1, GE)),
            _full((GE, GE)), _full((1, GE)),
            _full((GE, OF)), _full((1, OF)),
        ],
        out_specs=pl.BlockSpec((RP, OF), lambda i: (i, 0)),
        out_shape=jax.ShapeDtypeStruct((N, OF), jnp.float32),
    )(alo, ahi, x, r(lng[:HG]), r(lnb[:HG]), r(lng[HG:GE]), r(lnb[HG:GE]),
      r(lng[GE:]), r(lnb[GE:]),
      ppW1[:HG], ppW1[HG:GE], ppW1[GE:], r(ppb1), ppW2, r(ppb2), ppW3, r(ppb3))

    return out


# speedup vs baseline: 6.4880x; 1.0484x over previous
import jax
import jax.numpy as jnp
from jax import lax
from jax.experimental import pallas as pl
from jax.experimental.pallas import tpu as pltpu
from jax.experimental.pallas import tpu_sc as plsc

N = 10000
E = 320000
NF, EF, NE, EE, GE, OF = 128, 16, 16, 16, 32, 128
HG = GE // 2

NC, NS, L = 2, 16, 16
NW = NC * NS
EPW = E // NW
GCH = 2000
SCH = 2000
NPS = N // NS

P = 8
EP = E // P


def _leaky(v):
    return jnp.maximum(v, 0.01 * v)


def _node_body(x_ref, w1, b1, w2, b2, wx, o_ref):
    h = _leaky(jnp.dot(x_ref[...], w1[...], preferred_element_type=jnp.float32) + b1[...])
    h = _leaky(jnp.dot(h, w2[...], preferred_element_type=jnp.float32) + b2[...])
    o_ref[...] = jnp.dot(h, wx[...], preferred_element_type=jnp.float32)


def _msg_body(ea_ref, xp_ref, w1, b1, w2, b2, we, gb1, gw2l, gb2l, gw2h, gb2h,
              olo_ref, ohi_ref):
    ea = jnp.concatenate([ea_ref[:, j, :] for j in range(P)], axis=1)
    h = _leaky(jnp.dot(ea, w1[...], preferred_element_type=jnp.float32) + b1[...])
    h = _leaky(jnp.dot(h, w2[...], preferred_element_type=jnp.float32) + b2[...])
    h1 = _leaky(jnp.dot(h, we[...], preferred_element_type=jnp.float32)
                + xp_ref[...] + gb1[...])
    olo_ref[...] = _leaky(jnp.dot(h1, gw2l[...], preferred_element_type=jnp.float32) + gb2l[...])
    ohi_ref[...] = _leaky(jnp.dot(h1, gw2h[...], preferred_element_type=jnp.float32) + gb2h[...])


def _post_body(alo_ref, ahi_ref, x_ref, lng_l, lnb_l, lng_h, lnb_h,
               lng_x, lnb_x, w1l, w1h, w1x, b1, w2, b2, w3, b3, o_ref):
    al = alo_ref[0] + alo_ref[1]
    ah = ahi_ref[0] + ahi_ref[1]
    xb = x_ref[...]
    denom = float(GE + NF)
    mu = (jnp.sum(al, axis=1, keepdims=True) + jnp.sum(ah, axis=1, keepdims=True)
          + jnp.sum(xb, axis=1, keepdims=True)) / denom
    dl = al - mu
    dh = ah - mu
    dx = xb - mu
    var = (jnp.sum(dl * dl, axis=1, keepdims=True)
           + jnp.sum(dh * dh, axis=1, keepdims=True)
           + jnp.sum(dx * dx, axis=1, keepdims=True)) / denom
    inv = lax.rsqrt(var + 1e-5)
    hl = dl * inv * lng_l[...] + lnb_l[...]
    hh = dh * inv * lng_h[...] + lnb_h[...]
    hx = dx * inv * lng_x[...] + lnb_x[...]
    h = _leaky(jnp.dot(hl, w1l[...], preferred_element_type=jnp.float32)
               + jnp.dot(hh, w1h[...], preferred_element_type=jnp.float32)
               + jnp.dot(hx, w1x[...], preferred_element_type=jnp.float32) + b1[...])
    h = _leaky(jnp.dot(h, w2[...], preferred_element_type=jnp.float32) + b2[...])
    o_ref[...] = jnp.dot(h, w3[...], preferred_element_type=jnp.float32) + b3[...]


def _full(shape):
    return pl.BlockSpec(shape, lambda i: (0,) * len(shape))


def _gather_body(tab_hbm, eidx_hbm, out_hbm, idx_v, rows_v, sem):
    wid = lax.axis_index("s") * NC + lax.axis_index("c")
    base = wid * EPW

    def body(i, carry):
        off = pl.multiple_of(base + i * GCH, 8)
        pltpu.sync_copy(eidx_hbm.at[0, pl.ds(off, GCH)], idx_v)
        pltpu.async_copy(tab_hbm.at[idx_v], rows_v, sem).wait()
        pltpu.sync_copy(rows_v, out_hbm.at[pl.ds(off, GCH)])
        return carry

    lax.fori_loop(0, EPW // GCH, body, 0)


def _scatter_body(mlo_hbm, mhi_hbm, eidx_hbm, zeros_hbm, olo_hbm, ohi_hbm,
                  idx_v, mlo_v, mhi_v, tmp_v, acclo_sh, acchi_sh, sem):
    cid = lax.axis_index("c")
    sid = lax.axis_index("s")
    wid = sid * NC + cid
    pltpu.sync_copy(zeros_hbm.at[pl.ds(0, NPS)], tmp_v)
    pltpu.sync_copy(tmp_v, acclo_sh.at[pl.ds(sid * NPS, NPS)])
    pltpu.sync_copy(tmp_v, acchi_sh.at[pl.ds(sid * NPS, NPS)])
    plsc.subcore_barrier()
    base = wid * EPW

    def body(i, carry):
        off = pl.multiple_of(base + i * SCH, 8)
        pltpu.sync_copy(eidx_hbm.at[1, pl.ds(off, SCH)], idx_v)
        pltpu.sync_copy(mlo_hbm.at[pl.ds(off, SCH)], mlo_v)
        pltpu.sync_copy(mhi_hbm.at[pl.ds(off, SCH)], mhi_v)
        pltpu.sync_copy(mlo_v, acclo_sh.at[idx_v], add=True)
        pltpu.sync_copy(mhi_v, acchi_sh.at[idx_v], add=True)
        return carry

    lax.fori_loop(0, EPW // SCH, body, 0)
    plsc.subcore_barrier()
    pltpu.sync_copy(acclo_sh.at[pl.ds(sid * NPS, NPS)], tmp_v)
    pltpu.sync_copy(tmp_v, olo_hbm.at[cid, pl.ds(sid * NPS, NPS)])
    pltpu.sync_copy(acchi_sh.at[pl.ds(sid * NPS, NPS)], tmp_v)
    pltpu.sync_copy(tmp_v, ohi_hbm.at[cid, pl.ds(sid * NPS, NPS)])


def _sc_gather(node_pre, edge_index):
    call = pl.kernel(
        _gather_body,
        out_type=jax.ShapeDtypeStruct((E, NE), jnp.float32),
        mesh=plsc.VectorSubcoreMesh(core_axis_name="c", subcore_axis_name="s",
                                    num_cores=NC, num_subcores=NS),
        scratch_types=[
            pltpu.VMEM((GCH,), jnp.int32),
            pltpu.VMEM((GCH, NE), jnp.float32),
            pltpu.SemaphoreType.DMA,
        ],
        compiler_params=pltpu.CompilerParams(use_tc_tiling_on_sc=False),
    )
    return call(node_pre, edge_index)


def _sc_scatter(msg_lo, msg_hi, edge_index, zeros):
    call = pl.kernel(
        _scatter_body,
        out_type=(jax.ShapeDtypeStruct((NC, N, HG), jnp.float32),
                  jax.ShapeDtypeStruct((NC, N, HG), jnp.float32)),
        mesh=plsc.VectorSubcoreMesh(core_axis_name="c", subcore_axis_name="s",
                                    num_cores=NC, num_subcores=NS),
        scratch_types=[
            pltpu.VMEM((SCH,), jnp.int32),
            pltpu.VMEM((SCH, HG), jnp.float32),
            pltpu.VMEM((SCH, HG), jnp.float32),
            pltpu.VMEM((NPS, HG), jnp.float32),
            pltpu.VMEM_SHARED((N, HG), jnp.float32),
            pltpu.VMEM_SHARED((N, HG), jnp.float32),
            pltpu.SemaphoreType.DMA,
        ],
        compiler_params=pltpu.CompilerParams(use_tc_tiling_on_sc=False),
    )
    return call(msg_lo, msg_hi, edge_index, zeros)


def kernel(x, edge_index, edge_attr, neW1, neb1, neW2, neb2, eeW1, eeb1,
           eeW2, eeb2, gW1, gb1, gW2, gb2, lng, lnb, ppW1, ppb1, ppW2, ppb2,
           ppW3, ppb3):
    eidx = edge_index.astype(jnp.int32)
    gW1e = gW1[:EE]
    gW1x = gW1[EE:]
    r = lambda v: v.reshape(1, -1)

    RN = 2000
    node_pre = pl.pallas_call(
        _node_body,
        grid=(N // RN,),
        in_specs=[
            pl.BlockSpec((RN, NF), lambda i: (i, 0)),
            _full((NF, NE)), _full((1, NE)), _full((NE, NE)), _full((1, NE)),
            _full((NE, NE)),
        ],
        out_specs=pl.BlockSpec((RN, NE), lambda i: (i, 0)),
        out_shape=jax.ShapeDtypeStruct((N, NE), jnp.float32),
    )(x, neW1, r(neb1), neW2, r(neb2), gW1x)

    x_pre = _sc_gather(node_pre, eidx)

    eye = jnp.eye(P, dtype=jnp.float32)
    kb = lambda w: jnp.kron(eye, w)
    tb = lambda b: jnp.tile(b, P).reshape(1, -1)
    xp_p = x_pre.reshape(EP, P * NE)

    REP = 2000
    mlo_p, mhi_p = pl.pallas_call(
        _msg_body,
        grid=(EP // REP,),
        in_specs=[
            pl.BlockSpec((REP, P, EF), lambda i: (i, 0, 0)),
            pl.BlockSpec((REP, P * NE), lambda i: (i, 0)),
            _full((P * EF, P * EE)), _full((1, P * EE)),
            _full((P * EE, P * EE)), _full((1, P * EE)),
            _full((P * EE, P * NE)), _full((1, P * NE)),
            _full((P * NE, P * HG)), _full((1, P * HG)),
            _full((P * NE, P * HG)), _full((1, P * HG)),
        ],
        out_specs=(pl.BlockSpec((REP, P * HG), lambda i: (i, 0)),
                   pl.BlockSpec((REP, P * HG), lambda i: (i, 0))),
        out_shape=(jax.ShapeDtypeStruct((EP, P * HG), jnp.float32),
                   jax.ShapeDtypeStruct((EP, P * HG), jnp.float32)),
    )(edge_attr.reshape(EP, P, EF), xp_p, kb(eeW1), tb(eeb1), kb(eeW2), tb(eeb2),
      kb(gW1e), tb(gb1), kb(gW2[:, :HG]), tb(gb2[:HG]),
      kb(gW2[:, HG:]), tb(gb2[HG:]))

    zeros = jnp.zeros((NPS, HG), dtype=jnp.float32)
    alo, ahi = _sc_scatter(mlo_p.reshape(E, HG), mhi_p.reshape(E, HG),
                           eidx, zeros)

    RP = 2000
    out = pl.pallas_call(
        _post_body,
        grid=(N // RP,),
        in_specs=[
            pl.BlockSpec((NC, RP, HG), lambda i: (0, i, 0)),
            pl.BlockSpec((NC, RP, HG), lambda i: (0, i, 0)),
            pl.BlockSpec((RP, NF), lambda i: (i, 0)),
            _full((1, HG)), _full((1, HG)), _full((1, HG)), _full((1, HG)),
            _full((1, NF)), _full((1, NF)),
            _full((HG, GE)), _full((HG, GE)), _full((NF, GE)), _full((1, GE)),
            _full((GE, GE)), _full((1, GE)),
            _full((GE, OF)), _full((1, OF)),
        ],
        out_specs=pl.BlockSpec((RP, OF), lambda i: (i, 0)),
        out_shape=jax.ShapeDtypeStruct((N, OF), jnp.float32),
    )(alo, ahi, x, r(lng[:HG]), r(lnb[:HG]), r(lng[HG:GE]), r(lnb[HG:GE]),
      r(lng[GE:]), r(lnb[GE:]),
      ppW1[:HG], ppW1[HG:GE], ppW1[GE:], r(ppb1), ppW2, r(ppb2), ppW3, r(ppb3))

    return out


# speedup vs baseline: 6.8802x; 1.0604x over previous
import jax
import jax.numpy as jnp
from jax import lax
from jax.experimental import pallas as pl
from jax.experimental.pallas import tpu as pltpu
from jax.experimental.pallas import tpu_sc as plsc

N = 10000
E = 320000
NF, EF, NE, EE, GE, OF = 128, 16, 16, 16, 32, 128
HG = GE // 2

NC, NS, L = 2, 16, 16
NW = NC * NS
EPW = E // NW
GCH = 2000
SCH = 1000
NPS = N // NS

P = 8
EP = E // P


def _leaky(v):
    return jnp.maximum(v, 0.01 * v)


def _node_body(x_ref, w1, b1, w2, b2, wx, o_ref):
    h = _leaky(jnp.dot(x_ref[...], w1[...], preferred_element_type=jnp.float32) + b1[...])
    h = _leaky(jnp.dot(h, w2[...], preferred_element_type=jnp.float32) + b2[...])
    o_ref[...] = jnp.dot(h, wx[...], preferred_element_type=jnp.float32)


def _msg_body(ea_ref, xp_ref, w1, b1, w2, b2, we, gb1, gw2l, gb2l, gw2h, gb2h,
              olo_ref, ohi_ref):
    ea = jnp.concatenate([ea_ref[:, j, :] for j in range(P)], axis=1)
    h = _leaky(jnp.dot(ea, w1[...], preferred_element_type=jnp.float32) + b1[...])
    h = _leaky(jnp.dot(h, w2[...], preferred_element_type=jnp.float32) + b2[...])
    h1 = _leaky(jnp.dot(h, we[...], preferred_element_type=jnp.float32)
                + xp_ref[...] + gb1[...])
    olo_ref[...] = _leaky(jnp.dot(h1, gw2l[...], preferred_element_type=jnp.float32) + gb2l[...])
    ohi_ref[...] = _leaky(jnp.dot(h1, gw2h[...], preferred_element_type=jnp.float32) + gb2h[...])


def _post_body(alo_ref, ahi_ref, x_ref, lng_l, lnb_l, lng_h, lnb_h,
               lng_x, lnb_x, w1l, w1h, w1x, b1, w2, b2, w3, b3, o_ref):
    al = alo_ref[0] + alo_ref[1]
    ah = ahi_ref[0] + ahi_ref[1]
    xb = x_ref[...]
    denom = float(GE + NF)
    mu = (jnp.sum(al, axis=1, keepdims=True) + jnp.sum(ah, axis=1, keepdims=True)
          + jnp.sum(xb, axis=1, keepdims=True)) / denom
    dl = al - mu
    dh = ah - mu
    dx = xb - mu
    var = (jnp.sum(dl * dl, axis=1, keepdims=True)
           + jnp.sum(dh * dh, axis=1, keepdims=True)
           + jnp.sum(dx * dx, axis=1, keepdims=True)) / denom
    inv = lax.rsqrt(var + 1e-5)
    hl = dl * inv * lng_l[...] + lnb_l[...]
    hh = dh * inv * lng_h[...] + lnb_h[...]
    hx = dx * inv * lng_x[...] + lnb_x[...]
    h = _leaky(jnp.dot(hl, w1l[...], preferred_element_type=jnp.float32)
               + jnp.dot(hh, w1h[...], preferred_element_type=jnp.float32)
               + jnp.dot(hx, w1x[...], preferred_element_type=jnp.float32) + b1[...])
    h = _leaky(jnp.dot(h, w2[...], preferred_element_type=jnp.float32) + b2[...])
    o_ref[...] = jnp.dot(h, w3[...], preferred_element_type=jnp.float32) + b3[...]


def _full(shape):
    return pl.BlockSpec(shape, lambda i: (0,) * len(shape))


def _gather_body(tab_hbm, eidx_hbm, out_hbm,
                 idx_v0, idx_v1, rows_v0, rows_v1,
                 semi0, semi1, semg0, semg1, semo0, semo1):
    wid = lax.axis_index("s") * NC + lax.axis_index("c")
    base = wid * EPW
    nch = EPW // GCH
    idx_v = (idx_v0, idx_v1)
    rows_v = (rows_v0, rows_v1)
    semi = (semi0, semi1)
    semg = (semg0, semg1)
    semo = (semo0, semo1)

    def off(k):
        return pl.multiple_of(base + k * GCH, 8)

    idx_h = [None] * nch
    out_h = [None] * nch
    idx_h[0] = pltpu.async_copy(eidx_hbm.at[pl.ds(off(0), GCH)], idx_v[0], semi[0])
    for k in range(nch):
        b = k % 2
        idx_h[k].wait()
        if k >= 2:
            out_h[k - 2].wait()
        gh = pltpu.async_copy(tab_hbm.at[idx_v[b]], rows_v[b], semg[b])
        if k + 1 < nch:
            idx_h[k + 1] = pltpu.async_copy(
                eidx_hbm.at[pl.ds(off(k + 1), GCH)], idx_v[1 - b], semi[1 - b])
        gh.wait()
        out_h[k] = pltpu.async_copy(rows_v[b], out_hbm.at[pl.ds(off(k), GCH)], semo[b])
    out_h[nch - 2].wait()
    out_h[nch - 1].wait()


def _scatter_body(mlo_hbm, mhi_hbm, eidx_hbm, zeros_hbm, olo_hbm, ohi_hbm,
                  idx_v0, idx_v1, mlo_v0, mlo_v1, mhi_v0, mhi_v1, tmp_v,
                  acclo_sh, acchi_sh,
                  semi0, semi1, seml0, seml1, semh0, semh1):
    cid = lax.axis_index("c")
    sid = lax.axis_index("s")
    wid = sid * NC + cid
    pltpu.sync_copy(zeros_hbm.at[pl.ds(0, NPS)], tmp_v)
    pltpu.sync_copy(tmp_v, acclo_sh.at[pl.ds(sid * NPS, NPS)])
    pltpu.sync_copy(tmp_v, acchi_sh.at[pl.ds(sid * NPS, NPS)])
    plsc.subcore_barrier()
    base = wid * EPW
    nch = EPW // SCH
    idx_v = (idx_v0, idx_v1)
    mlo_v = (mlo_v0, mlo_v1)
    mhi_v = (mhi_v0, mhi_v1)
    semi = (semi0, semi1)
    seml = (seml0, seml1)
    semh = (semh0, semh1)

    def off(k):
        return pl.multiple_of(base + k * SCH, 8)

    def issue(k):
        b = k % 2
        return (pltpu.async_copy(eidx_hbm.at[pl.ds(E + off(k), SCH)], idx_v[b], semi[b]),
                pltpu.async_copy(mlo_hbm.at[pl.ds(off(k), SCH)], mlo_v[b], seml[b]),
                pltpu.async_copy(mhi_hbm.at[pl.ds(off(k), SCH)], mhi_v[b], semh[b]))

    h = issue(0)
    for k in range(nch):
        b = k % 2
        nxt = issue(k + 1) if k + 1 < nch else None
        for hh in h:
            hh.wait()
        pltpu.sync_copy(mlo_v[b], acclo_sh.at[idx_v[b]], add=True)
        pltpu.sync_copy(mhi_v[b], acchi_sh.at[idx_v[b]], add=True)
        h = nxt
    plsc.subcore_barrier()
    pltpu.sync_copy(acclo_sh.at[pl.ds(sid * NPS, NPS)], tmp_v)
    pltpu.sync_copy(tmp_v, olo_hbm.at[cid, pl.ds(sid * NPS, NPS)])
    pltpu.sync_copy(acchi_sh.at[pl.ds(sid * NPS, NPS)], tmp_v)
    pltpu.sync_copy(tmp_v, ohi_hbm.at[cid, pl.ds(sid * NPS, NPS)])


def _sc_gather(node_pre, edge_index):
    call = pl.kernel(
        _gather_body,
        out_type=jax.ShapeDtypeStruct((E, NE), jnp.float32),
        mesh=plsc.VectorSubcoreMesh(core_axis_name="c", subcore_axis_name="s",
                                    num_cores=NC, num_subcores=NS),
        scratch_types=[
            pltpu.VMEM((GCH,), jnp.int32),
            pltpu.VMEM((GCH,), jnp.int32),
            pltpu.VMEM((GCH, NE), jnp.float32),
            pltpu.VMEM((GCH, NE), jnp.float32),
            pltpu.SemaphoreType.DMA,
            pltpu.SemaphoreType.DMA,
            pltpu.SemaphoreType.DMA,
            pltpu.SemaphoreType.DMA,
            pltpu.SemaphoreType.DMA,
            pltpu.SemaphoreType.DMA,
        ],
        compiler_params=pltpu.CompilerParams(use_tc_tiling_on_sc=False),
    )
    return call(node_pre, edge_index)


def _sc_scatter(msg_lo, msg_hi, edge_index, zeros):
    call = pl.kernel(
        _scatter_body,
        out_type=(jax.ShapeDtypeStruct((NC, N, HG), jnp.float32),
                  jax.ShapeDtypeStruct((NC, N, HG), jnp.float32)),
        mesh=plsc.VectorSubcoreMesh(core_axis_name="c", subcore_axis_name="s",
                                    num_cores=NC, num_subcores=NS),
        scratch_types=[
            pltpu.VMEM((SCH,), jnp.int32),
            pltpu.VMEM((SCH,), jnp.int32),
            pltpu.VMEM((SCH, HG), jnp.float32),
            pltpu.VMEM((SCH, HG), jnp.float32),
            pltpu.VMEM((SCH, HG), jnp.float32),
            pltpu.VMEM((SCH, HG), jnp.float32),
            pltpu.VMEM((NPS, HG), jnp.float32),
            pltpu.VMEM_SHARED((N, HG), jnp.float32),
            pltpu.VMEM_SHARED((N, HG), jnp.float32),
            pltpu.SemaphoreType.DMA,
            pltpu.SemaphoreType.DMA,
            pltpu.SemaphoreType.DMA,
            pltpu.SemaphoreType.DMA,
            pltpu.SemaphoreType.DMA,
            pltpu.SemaphoreType.DMA,
        ],
        compiler_params=pltpu.CompilerParams(use_tc_tiling_on_sc=False),
    )
    return call(msg_lo, msg_hi, edge_index, zeros)


def kernel(x, edge_index, edge_attr, neW1, neb1, neW2, neb2, eeW1, eeb1,
           eeW2, eeb2, gW1, gb1, gW2, gb2, lng, lnb, ppW1, ppb1, ppW2, ppb2,
           ppW3, ppb3):
    eidx = edge_index.astype(jnp.int32).reshape(2 * E)
    gW1e = gW1[:EE]
    gW1x = gW1[EE:]
    r = lambda v: v.reshape(1, -1)

    RN = 2000
    node_pre = pl.pallas_call(
        _node_body,
        grid=(N // RN,),
        in_specs=[
            pl.BlockSpec((RN, NF), lambda i: (i, 0)),
            _full((NF, NE)), _full((1, NE)), _full((NE, NE)), _full((1, NE)),
            _full((NE, NE)),
        ],
        out_specs=pl.BlockSpec((RN, NE), lambda i: (i, 0)),
        out_shape=jax.ShapeDtypeStruct((N, NE), jnp.float32),
    )(x, neW1, r(neb1), neW2, r(neb2), gW1x)

    x_pre = _sc_gather(node_pre, eidx)

    eye = jnp.eye(P, dtype=jnp.float32)
    kb = lambda w: jnp.kron(eye, w)
    tb = lambda b: jnp.tile(b, P).reshape(1, -1)
    xp_p = x_pre.reshape(EP, P * NE)

    REP = 2000
    mlo_p, mhi_p = pl.pallas_call(
        _msg_body,
        grid=(EP // REP,),
        in_specs=[
            pl.BlockSpec((REP, P, EF), lambda i: (i, 0, 0)),
            pl.BlockSpec((REP, P * NE), lambda i: (i, 0)),
            _full((P * EF, P * EE)), _full((1, P * EE)),
            _full((P * EE, P * EE)), _full((1, P * EE)),
            _full((P * EE, P * NE)), _full((1, P * NE)),
            _full((P * NE, P * HG)), _full((1, P * HG)),
            _full((P * NE, P * HG)), _full((1, P * HG)),
        ],
        out_specs=(pl.BlockSpec((REP, P * HG), lambda i: (i, 0)),
                   pl.BlockSpec((REP, P * HG), lambda i: (i, 0))),
        out_shape=(jax.ShapeDtypeStruct((EP, P * HG), jnp.float32),
                   jax.ShapeDtypeStruct((EP, P * HG), jnp.float32)),
    )(edge_attr.reshape(EP, P, EF), xp_p, kb(eeW1), tb(eeb1), kb(eeW2), tb(eeb2),
      kb(gW1e), tb(gb1), kb(gW2[:, :HG]), tb(gb2[:HG]),
      kb(gW2[:, HG:]), tb(gb2[HG:]))

    zeros = jnp.zeros((NPS, HG), dtype=jnp.float32)
    alo, ahi = _sc_scatter(mlo_p.reshape(E, HG), mhi_p.reshape(E, HG),
                           eidx, zeros)

    RP = 2000
    out = pl.pallas_call(
        _post_body,
        grid=(N // RP,),
        in_specs=[
            pl.BlockSpec((NC, RP, HG), lambda i: (0, i, 0)),
            pl.BlockSpec((NC, RP, HG), lambda i: (0, i, 0)),
            pl.BlockSpec((RP, NF), lambda i: (i, 0)),
            _full((1, HG)), _full((1, HG)), _full((1, HG)), _full((1, HG)),
            _full((1, NF)), _full((1, NF)),
            _full((HG, GE)), _full((HG, GE)), _full((NF, GE)), _full((1, GE)),
            _full((GE, GE)), _full((1, GE)),
            _full((GE, OF)), _full((1, OF)),
        ],
        out_specs=pl.BlockSpec((RP, OF), lambda i: (i, 0)),
        out_shape=jax.ShapeDtypeStruct((N, OF), jnp.float32),
    )(alo, ahi, x, r(lng[:HG]), r(lnb[:HG]), r(lng[HG:GE]), r(lnb[HG:GE]),
      r(lng[GE:]), r(lnb[GE:]),
      ppW1[:HG], ppW1[HG:GE], ppW1[GE:], r(ppb1), ppW2, r(ppb2), ppW3, r(ppb3))

    return out


# speedup vs baseline: 7.0540x; 1.0253x over previous
import jax
import jax.numpy as jnp
from jax import lax
from jax.experimental import pallas as pl
from jax.experimental.pallas import tpu as pltpu
from jax.experimental.pallas import tpu_sc as plsc

N = 10000
E = 320000
NF, EF, NE, EE, GE, OF = 128, 16, 16, 16, 32, 128
HG = GE // 2

NC, NS, L = 2, 16, 16
NW = NC * NS
EPW = E // NW
GCH = 2000
SCH = 1000
NPS = N // NS

P = 8
EP = E // P
EH = E // 2
EPW2 = EH // NW
GCH2 = 1000
SCH2 = 1000


def _leaky(v):
    return jnp.maximum(v, 0.01 * v)


def _node_body(x_ref, w1, b1, w2, b2, wx, o_ref):
    h = _leaky(jnp.dot(x_ref[...], w1[...], preferred_element_type=jnp.float32) + b1[...])
    h = _leaky(jnp.dot(h, w2[...], preferred_element_type=jnp.float32) + b2[...])
    o_ref[...] = jnp.dot(h, wx[...], preferred_element_type=jnp.float32)


def _msg_body(ea_ref, xp_ref, w1, b1, w2, b2, we, gb1, gw2l, gb2l, gw2h, gb2h,
              olo_ref, ohi_ref):
    ea = jnp.concatenate([ea_ref[:, j, :] for j in range(P)], axis=1)
    h = _leaky(jnp.dot(ea, w1[...], preferred_element_type=jnp.float32) + b1[...])
    h = _leaky(jnp.dot(h, w2[...], preferred_element_type=jnp.float32) + b2[...])
    h1 = _leaky(jnp.dot(h, we[...], preferred_element_type=jnp.float32)
                + xp_ref[...] + gb1[...])
    olo_ref[...] = _leaky(jnp.dot(h1, gw2l[...], preferred_element_type=jnp.float32) + gb2l[...])
    ohi_ref[...] = _leaky(jnp.dot(h1, gw2h[...], preferred_element_type=jnp.float32) + gb2h[...])


def _post_body(alo_ref, ahi_ref, x_ref, lng_l, lnb_l, lng_h, lnb_h,
               lng_x, lnb_x, w1l, w1h, w1x, b1, w2, b2, w3, b3, o_ref):
    al = alo_ref[0] + alo_ref[1]
    ah = ahi_ref[0] + ahi_ref[1]
    xb = x_ref[...]
    denom = float(GE + NF)
    mu = (jnp.sum(al, axis=1, keepdims=True) + jnp.sum(ah, axis=1, keepdims=True)
          + jnp.sum(xb, axis=1, keepdims=True)) / denom
    dl = al - mu
    dh = ah - mu
    dx = xb - mu
    var = (jnp.sum(dl * dl, axis=1, keepdims=True)
           + jnp.sum(dh * dh, axis=1, keepdims=True)
           + jnp.sum(dx * dx, axis=1, keepdims=True)) / denom
    inv = lax.rsqrt(var + 1e-5)
    hl = dl * inv * lng_l[...] + lnb_l[...]
    hh = dh * inv * lng_h[...] + lnb_h[...]
    hx = dx * inv * lng_x[...] + lnb_x[...]
    h = _leaky(jnp.dot(hl, w1l[...], preferred_element_type=jnp.float32)
               + jnp.dot(hh, w1h[...], preferred_element_type=jnp.float32)
               + jnp.dot(hx, w1x[...], preferred_element_type=jnp.float32) + b1[...])
    h = _leaky(jnp.dot(h, w2[...], preferred_element_type=jnp.float32) + b2[...])
    o_ref[...] = jnp.dot(h, w3[...], preferred_element_type=jnp.float32) + b3[...]


def _full(shape):
    return pl.BlockSpec(shape, lambda i: (0,) * len(shape))


def _make_gather_body(ebase):
  def _gather_body(tab_hbm, eidx_hbm, out_hbm,
                 idx_v0, idx_v1, rows_v0, rows_v1,
                 semi0, semi1, semg0, semg1, semo0, semo1):
    wid = lax.axis_index("s") * NC + lax.axis_index("c")
    base = wid * EPW2
    nch = EPW2 // GCH2
    idx_v = (idx_v0, idx_v1)
    rows_v = (rows_v0, rows_v1)
    semi = (semi0, semi1)
    semg = (semg0, semg1)
    semo = (semo0, semo1)

    def off(k):
        return pl.multiple_of(base + k * GCH2, 8)

    idx_h = [None] * nch
    out_h = [None] * nch
    idx_h[0] = pltpu.async_copy(eidx_hbm.at[pl.ds(ebase + off(0), GCH2)], idx_v[0], semi[0])
    for k in range(nch):
        b = k % 2
        idx_h[k].wait()
        if k >= 2:
            out_h[k - 2].wait()
        gh = pltpu.async_copy(tab_hbm.at[idx_v[b]], rows_v[b], semg[b])
        if k + 1 < nch:
            idx_h[k + 1] = pltpu.async_copy(
                eidx_hbm.at[pl.ds(ebase + off(k + 1), GCH2)], idx_v[1 - b], semi[1 - b])
        gh.wait()
        out_h[k] = pltpu.async_copy(rows_v[b], out_hbm.at[pl.ds(off(k), GCH2)], semo[b])
    out_h[nch - 2].wait()
    out_h[nch - 1].wait()
  return _gather_body


def _make_scatter_body(ebase, seeded):
  def _scatter_body(mlo_hbm, mhi_hbm, eidx_hbm, ilo_hbm, ihi_hbm, olo_hbm, ohi_hbm,
                  idx_v0, idx_v1, mlo_v0, mlo_v1, mhi_v0, mhi_v1, tmp_v,
                  acclo_sh, acchi_sh,
                  semi0, semi1, seml0, seml1, semh0, semh1):
    cid = lax.axis_index("c")
    sid = lax.axis_index("s")
    wid = sid * NC + cid
    if seeded:
        pltpu.sync_copy(ilo_hbm.at[cid, pl.ds(sid * NPS, NPS)], tmp_v)
        pltpu.sync_copy(tmp_v, acclo_sh.at[pl.ds(sid * NPS, NPS)])
        pltpu.sync_copy(ihi_hbm.at[cid, pl.ds(sid * NPS, NPS)], tmp_v)
        pltpu.sync_copy(tmp_v, acchi_sh.at[pl.ds(sid * NPS, NPS)])
    else:
        pltpu.sync_copy(ilo_hbm.at[pl.ds(0, NPS)], tmp_v)
        pltpu.sync_copy(tmp_v, acclo_sh.at[pl.ds(sid * NPS, NPS)])
        pltpu.sync_copy(tmp_v, acchi_sh.at[pl.ds(sid * NPS, NPS)])
    plsc.subcore_barrier()
    base = wid * EPW2
    nch = EPW2 // SCH2
    idx_v = (idx_v0, idx_v1)
    mlo_v = (mlo_v0, mlo_v1)
    mhi_v = (mhi_v0, mhi_v1)
    semi = (semi0, semi1)
    seml = (seml0, seml1)
    semh = (semh0, semh1)

    def off(k):
        return pl.multiple_of(base + k * SCH2, 8)

    def issue(k):
        b = k % 2
        return (pltpu.async_copy(eidx_hbm.at[pl.ds(E + ebase + off(k), SCH2)], idx_v[b], semi[b]),
                pltpu.async_copy(mlo_hbm.at[pl.ds(off(k), SCH2)], mlo_v[b], seml[b]),
                pltpu.async_copy(mhi_hbm.at[pl.ds(off(k), SCH2)], mhi_v[b], semh[b]))

    h = issue(0)
    for k in range(nch):
        b = k % 2
        nxt = issue(k + 1) if k + 1 < nch else None
        for hh in h:
            hh.wait()
        pltpu.sync_copy(mlo_v[b], acclo_sh.at[idx_v[b]], add=True)
        pltpu.sync_copy(mhi_v[b], acchi_sh.at[idx_v[b]], add=True)
        h = nxt
    plsc.subcore_barrier()
    pltpu.sync_copy(acclo_sh.at[pl.ds(sid * NPS, NPS)], tmp_v)
    pltpu.sync_copy(tmp_v, olo_hbm.at[cid, pl.ds(sid * NPS, NPS)])
    pltpu.sync_copy(acchi_sh.at[pl.ds(sid * NPS, NPS)], tmp_v)
    pltpu.sync_copy(tmp_v, ohi_hbm.at[cid, pl.ds(sid * NPS, NPS)])
  return _scatter_body


def _sc_gather(node_pre, edge_index, ebase):
    call = pl.kernel(
        _make_gather_body(ebase),
        out_type=jax.ShapeDtypeStruct((EH, NE), jnp.float32),
        mesh=plsc.VectorSubcoreMesh(core_axis_name="c", subcore_axis_name="s",
                                    num_cores=NC, num_subcores=NS),
        scratch_types=[
            pltpu.VMEM((GCH2,), jnp.int32),
            pltpu.VMEM((GCH2,), jnp.int32),
            pltpu.VMEM((GCH2, NE), jnp.float32),
            pltpu.VMEM((GCH2, NE), jnp.float32),
            pltpu.SemaphoreType.DMA,
            pltpu.SemaphoreType.DMA,
            pltpu.SemaphoreType.DMA,
            pltpu.SemaphoreType.DMA,
            pltpu.SemaphoreType.DMA,
            pltpu.SemaphoreType.DMA,
        ],
        compiler_params=pltpu.CompilerParams(use_tc_tiling_on_sc=False),
    )
    return call(node_pre, edge_index)


def _sc_scatter(msg_lo, msg_hi, edge_index, init_lo, init_hi, ebase, seeded):
    call = pl.kernel(
        _make_scatter_body(ebase, seeded),
        out_type=(jax.ShapeDtypeStruct((NC, N, HG), jnp.float32),
                  jax.ShapeDtypeStruct((NC, N, HG), jnp.float32)),
        mesh=plsc.VectorSubcoreMesh(core_axis_name="c", subcore_axis_name="s",
                                    num_cores=NC, num_subcores=NS),
        scratch_types=[
            pltpu.VMEM((SCH2,), jnp.int32),
            pltpu.VMEM((SCH2,), jnp.int32),
            pltpu.VMEM((SCH2, HG), jnp.float32),
            pltpu.VMEM((SCH2, HG), jnp.float32),
            pltpu.VMEM((SCH2, HG), jnp.float32),
            pltpu.VMEM((SCH2, HG), jnp.float32),
            pltpu.VMEM((NPS, HG), jnp.float32),
            pltpu.VMEM_SHARED((N, HG), jnp.float32),
            pltpu.VMEM_SHARED((N, HG), jnp.float32),
            pltpu.SemaphoreType.DMA,
            pltpu.SemaphoreType.DMA,
            pltpu.SemaphoreType.DMA,
            pltpu.SemaphoreType.DMA,
            pltpu.SemaphoreType.DMA,
            pltpu.SemaphoreType.DMA,
        ],
        compiler_params=pltpu.CompilerParams(use_tc_tiling_on_sc=False),
    )
    return call(msg_lo, msg_hi, edge_index, init_lo, init_hi)


def kernel(x, edge_index, edge_attr, neW1, neb1, neW2, neb2, eeW1, eeb1,
           eeW2, eeb2, gW1, gb1, gW2, gb2, lng, lnb, ppW1, ppb1, ppW2, ppb2,
           ppW3, ppb3):
    eidx = edge_index.astype(jnp.int32).reshape(2 * E)
    gW1e = gW1[:EE]
    gW1x = gW1[EE:]
    r = lambda v: v.reshape(1, -1)

    RN = 2000
    node_pre = pl.pallas_call(
        _node_body,
        grid=(N // RN,),
        in_specs=[
            pl.BlockSpec((RN, NF), lambda i: (i, 0)),
            _full((NF, NE)), _full((1, NE)), _full((NE, NE)), _full((1, NE)),
            _full((NE, NE)),
        ],
        out_specs=pl.BlockSpec((RN, NE), lambda i: (i, 0)),
        out_shape=jax.ShapeDtypeStruct((N, NE), jnp.float32),
    )(x, neW1, r(neb1), neW2, r(neb2), gW1x)

    eye = jnp.eye(P, dtype=jnp.float32)
    kb = lambda w: jnp.kron(eye, w)
    tb = lambda b: jnp.tile(b, P).reshape(1, -1)
    ea3 = edge_attr.reshape(EP, P, EF)
    EPH = EH // P
    REP = 2000
    nblk = EPH // REP

    def msg_half(xp_half, half):
        xp_p = xp_half.reshape(EPH, P * NE)
        boff = half * nblk
        return pl.pallas_call(
            _msg_body,
            grid=(nblk,),
            in_specs=[
                pl.BlockSpec((REP, P, EF), lambda i: (i + boff, 0, 0)),
                pl.BlockSpec((REP, P * NE), lambda i: (i, 0)),
                _full((P * EF, P * EE)), _full((1, P * EE)),
                _full((P * EE, P * EE)), _full((1, P * EE)),
                _full((P * EE, P * NE)), _full((1, P * NE)),
                _full((P * NE, P * HG)), _full((1, P * HG)),
                _full((P * NE, P * HG)), _full((1, P * HG)),
            ],
            out_specs=(pl.BlockSpec((REP, P * HG), lambda i: (i, 0)),
                       pl.BlockSpec((REP, P * HG), lambda i: (i, 0))),
            out_shape=(jax.ShapeDtypeStruct((EPH, P * HG), jnp.float32),
                       jax.ShapeDtypeStruct((EPH, P * HG), jnp.float32)),
        )(ea3, xp_p, kb(eeW1), tb(eeb1), kb(eeW2), tb(eeb2),
          kb(gW1e), tb(gb1), kb(gW2[:, :HG]), tb(gb2[:HG]),
          kb(gW2[:, HG:]), tb(gb2[HG:]))

    zeros = jnp.zeros((NPS, HG), dtype=jnp.float32)
    xa = _sc_gather(node_pre, eidx, 0)
    xb = _sc_gather(node_pre, eidx, EH)
    mlo_a, mhi_a = msg_half(xa, 0)
    mlo_b, mhi_b = msg_half(xb, 1)
    plo, phi = _sc_scatter(mlo_a.reshape(EH, HG), mhi_a.reshape(EH, HG),
                           eidx, zeros, zeros, 0, False)
    alo, ahi = _sc_scatter(mlo_b.reshape(EH, HG), mhi_b.reshape(EH, HG),
                           eidx, plo, phi, EH, True)

    RP = 2000
    out = pl.pallas_call(
        _post_body,
        grid=(N // RP,),
        in_specs=[
            pl.BlockSpec((NC, RP, HG), lambda i: (0, i, 0)),
            pl.BlockSpec((NC, RP, HG), lambda i: (0, i, 0)),
            pl.BlockSpec((RP, NF), lambda i: (i, 0)),
            _full((1, HG)), _full((1, HG)), _full((1, HG)), _full((1, HG)),
            _full((1, NF)), _full((1, NF)),
            _full((HG, GE)), _full((HG, GE)), _full((NF, GE)), _full((1, GE)),
            _full((GE, GE)), _full((1, GE)),
            _full((GE, OF)), _full((1, OF)),
        ],
        out_specs=pl.BlockSpec((RP, OF), lambda i: (i, 0)),
        out_shape=jax.ShapeDtypeStruct((N, OF), jnp.float32),
    )(alo, ahi, x, r(lng[:HG]), r(lnb[:HG]), r(lng[HG:GE]), r(lnb[HG:GE]),
      r(lng[GE:]), r(lnb[GE:]),
      ppW1[:HG], ppW1[HG:GE], ppW1[GE:], r(ppb1), ppW2, r(ppb2), ppW3, r(ppb3))

    return out
